# baseline (speedup 1.0000x reference)
import functools

import jax
import jax.numpy as jnp
from jax import lax
from jax.experimental import pallas as pl
from jax.experimental.pallas import tpu as pltpu
from jax.experimental.pallas import tpu_sc as plsc

_D = 1024
_B = 4
_S = 2048
_N = _B * _S
_NW = 32
_RPW = _N // _NW
_RCH = 32
_NCH = _RPW // _RCH
_BLK = 128
_EPS = 1e-5


def _sc_gather_body(ids_hbm, tok_hbm, out_hbm, idx_v, buf0, buf1, sem0, sem1):
    cid = lax.axis_index("c")
    sid = lax.axis_index("s")
    wid = sid * 2 + cid
    base = wid * _RPW

    pltpu.sync_copy(ids_hbm.at[pl.ds(base, _RPW)], idx_v)

    bufs = (buf0, buf1)
    sems = (sem0, sem1)
    handles = [None, None]
    handles[0] = pltpu.async_copy(
        tok_hbm.at[idx_v.at[pl.ds(0, _RCH)]], bufs[0], sems[0])
    for k in range(_NCH):
        if k + 1 < _NCH:
            handles[(k + 1) % 2] = pltpu.async_copy(
                tok_hbm.at[idx_v.at[pl.ds((k + 1) * _RCH, _RCH)]],
                bufs[(k + 1) % 2], sems[(k + 1) % 2])
        handles[k % 2].wait()
        pltpu.sync_copy(bufs[k % 2], out_hbm.at[pl.ds(base + k * _RCH, _RCH)])


@functools.partial(
    pl.kernel,
    out_type=jax.ShapeDtypeStruct((_N, _D), jnp.float32),
    mesh=plsc.VectorSubcoreMesh(core_axis_name="c", subcore_axis_name="s"),
    scratch_types=[
        pltpu.VMEM((_RPW,), jnp.int32),
        pltpu.VMEM((_RCH, _D), jnp.float32),
        pltpu.VMEM((_RCH, _D), jnp.float32),
        pltpu.SemaphoreType.DMA,
        pltpu.SemaphoreType.DMA,
    ],
)
def _sc_gather(ids_hbm, tok_hbm, out_hbm, idx_v, buf0, buf1, sem0, sem1):
    _sc_gather_body(ids_hbm, tok_hbm, out_hbm, idx_v, buf0, buf1, sem0, sem1)


def _ln_body(x_ref, pos_ref, msk_ref, gam_ref, bet_ref, out_ref):
    x = x_ref[...] + pos_ref[...]
    mean = jnp.mean(x, axis=-1, keepdims=True)
    var = jnp.mean(x * x, axis=-1, keepdims=True) - mean * mean
    y = (x - mean) * lax.rsqrt(var + _EPS) * gam_ref[...] + bet_ref[...]
    out_ref[...] = y * msk_ref[...]


def _tc_layernorm(gathered, pos_emb, maskcol, gamma2, beta2):
    n_sblk = _S // _BLK
    return pl.pallas_call(
        _ln_body,
        grid=(n_sblk, _B),
        in_specs=[
            pl.BlockSpec((_BLK, _D), lambda i, j: (j * n_sblk + i, 0)),
            pl.BlockSpec((_BLK, _D), lambda i, j: (i, 0)),
            pl.BlockSpec((_BLK, 1), lambda i, j: (j * n_sblk + i, 0)),
            pl.BlockSpec((1, _D), lambda i, j: (0, 0)),
            pl.BlockSpec((1, _D), lambda i, j: (0, 0)),
        ],
        out_specs=pl.BlockSpec((_BLK, _D), lambda i, j: (j * n_sblk + i, 0)),
        out_shape=jax.ShapeDtypeStruct((_N, _D), jnp.float32),
    )(gathered, pos_emb, maskcol, gamma2, beta2)


@jax.jit
def kernel(input_ids, attention_mask, token_emb, pos_emb, ln_gamma, ln_beta):
    ids = input_ids.reshape(-1).astype(jnp.int32)
    gathered = _sc_gather(ids, token_emb)
    maskcol = attention_mask.astype(jnp.float32).reshape(_N, 1)
    out = _tc_layernorm(gathered, pos_emb,
                        maskcol, ln_gamma.reshape(1, _D), ln_beta.reshape(1, _D))
    return out.reshape(_B, _S, _D)
